# baseline (speedup 1.0000x reference)
import numpy as np
import jax
import jax.numpy as jnp
from jax import lax
from jax.experimental import pallas as pl
from jax.experimental.pallas import tpu as pltpu
from jax.experimental.pallas import tpu_sc as plsc

_BATCH = 4
_NPTS = 120000
_L = 16
_NC, _NS = 2, 16
_NW = _NC * _NS
_B = 384
_NBLK = _NPTS // _B
_TAIL_S = _NBLK * _B
_TAIL_B = _NPTS - _TAIL_S

_F32 = jnp.float32
_I32 = jnp.int32

_CX1 = np.float32(np.float32(200.0) * (np.float32(1.0) / np.float32(100.0)))
_CZ1 = np.float32(np.float32(30.0) * (np.float32(1.0) / np.float32(6.0)))


def _quant_vec(in_v, outs_v, b, j):
    (idx0_v, xy0_v, top0_v, idx20_v, xz0_v, front0_v,
     idx1_v, xy1_v, top1_v, idx21_v, xz1_v, front1_v) = outs_v
    sl = pl.ds(j * _L, _L)
    xv = in_v[b, 0, sl]
    yv = in_v[b, 1, sl]
    zv = in_v[b, 2, sl]
    fx1 = (xv + 50.0) * _CX1
    fy1 = (yv + 50.0) * _CX1
    fz1 = (zv + 4.0) * _CZ1
    fx0 = fx1 + fx1
    fy0 = fy1 + fy1
    fz0 = fz1 + fz1

    def emit(fx, fy, fz, sxy, sz, idx_v, xy_v, top_v, idx2_v, xz_v, front_v):
        ix = fx.astype(_I32)
        iy = fy.astype(_I32)
        iz = fz.astype(_I32)
        rx = fx - ix.astype(_F32)
        ry = fy - iy.astype(_F32)
        rz = fz - iz.astype(_F32)
        idx_v[b, sl] = ix * sxy + iy
        idx2_v[b, sl] = ix * (sxy * sz) + iy * sz + iz
        xy_v[b, 0, sl] = ix
        xy_v[b, 1, sl] = iy
        top_v[b, 0, sl] = rx
        top_v[b, 1, sl] = ry
        xz_v[0, b, sl] = ix
        xz_v[1, b, sl] = iy
        xz_v[2, b, sl] = iz
        front_v[b, 0, sl] = rx
        front_v[b, 1, sl] = rz

    emit(fx0, fy0, fz0, 400, 60, idx0_v, xy0_v, top0_v, idx20_v, xz0_v, front0_v)
    emit(fx1, fy1, fz1, 200, 30, idx1_v, xy1_v, top1_v, idx21_v, xz1_v, front1_v)


def _compute_block(in_v, outs_v, nvec):
    def it(j, c):
        b = j // nvec
        _quant_vec(in_v, outs_v, b, j - b * nvec)
        return c

    lax.fori_loop(0, _BATCH * nvec, it, None)


def _out_copies(pct_echo_hbm, outs_hbm, in_v, outs_v, sem, s, size):
    sl = pl.ds(s, size)
    copies = [pltpu.make_async_copy(in_v, pct_echo_hbm.at[:, :, sl], sem)]
    for ov, oh in zip(outs_v, outs_hbm):
        if oh.ndim == 2:
            copies.append(pltpu.make_async_copy(ov, oh.at[:, sl], sem))
        else:
            copies.append(pltpu.make_async_copy(ov, oh.at[:, :, sl], sem))
    return copies


def _sc_body(pct_hbm, *refs):
    pct_echo_hbm = refs[0]
    outs_hbm = refs[1:13]
    in_v = refs[13:15]
    outs_v = (refs[15:27], refs[27:39])
    in_t = refs[39]
    outs_t = refs[40:52]
    sem_in = refs[52:54]
    sem_out = refs[54:56]
    wid = lax.axis_index("s") * _NC + lax.axis_index("c")
    nblk_w = (_NBLK - 1 - wid) // _NW + 1

    def in_fire(i, p):
        s = (i * _NW + wid) * _B
        pltpu.async_copy(pct_hbm.at[:, :, pl.ds(s, _B)], in_v[p], sem_in[p])

    def out_copies(i, p):
        s = (i * _NW + wid) * _B
        return _out_copies(pct_echo_hbm, outs_hbm, in_v[p], outs_v[p],
                           sem_out[p], s, _B)

    in_fire(0, 0)

    def step(g, _):
        for p in range(2):
            i = g * 2 + p

            @pl.when(i < nblk_w)
            def _():
                @pl.when(i + 1 < nblk_w)
                def _():
                    in_fire(i + 1, 1 - p)

                @pl.when(i >= 2)
                def _():
                    for c in out_copies(i - 2, p):
                        c.wait()

                pltpu.make_async_copy(
                    pct_hbm.at[:, :, pl.ds(0, _B)], in_v[p], sem_in[p]).wait()
                _compute_block(in_v[p], outs_v[p], _B // _L)
                for c in out_copies(i, p):
                    c.start()

        return _

    lax.fori_loop(0, (_NBLK // _NW) // 2 + 1, step, None)

    for p in range(2):
        @pl.when((nblk_w >= 1) & ((nblk_w - 1) % 2 == p))
        def _():
            for c in out_copies(nblk_w - 1, p):
                c.wait()

        @pl.when((nblk_w >= 2) & ((nblk_w - 2) % 2 == p))
        def _():
            for c in out_copies(nblk_w - 2, p):
                c.wait()

    @pl.when(wid == _NW - 1)
    def _():
        sl = pl.ds(_TAIL_S, _TAIL_B)
        pltpu.async_copy(pct_hbm.at[:, :, sl], in_t, sem_in[0]).wait()
        _compute_block(in_t, outs_t, _TAIL_B // _L)
        copies = _out_copies(pct_echo_hbm, outs_hbm, in_t, outs_t,
                             sem_out[0], _TAIL_S, _TAIL_B)
        for c in copies:
            c.start()
        for c in copies:
            c.wait()


def _out_types(n):
    return [
        jax.ShapeDtypeStruct((_BATCH, n), _I32),
        jax.ShapeDtypeStruct((_BATCH, 2, n), _I32),
        jax.ShapeDtypeStruct((_BATCH, 2, n), _F32),
        jax.ShapeDtypeStruct((_BATCH, n), _I32),
        jax.ShapeDtypeStruct((3, _BATCH, n), _I32),
        jax.ShapeDtypeStruct((_BATCH, 2, n), _F32),
    ]


def _buf_types(n):
    return [
        pltpu.VMEM((_BATCH, n), _I32),
        pltpu.VMEM((_BATCH, 2, n), _I32),
        pltpu.VMEM((_BATCH, 2, n), _F32),
        pltpu.VMEM((_BATCH, n), _I32),
        pltpu.VMEM((3, _BATCH, n), _I32),
        pltpu.VMEM((_BATCH, 2, n), _F32),
    ]


def _run_sc(pct):
    out_type = ([jax.ShapeDtypeStruct((_BATCH, 4, _NPTS), _F32)]
                + _out_types(_NPTS) * 2)

    def bufs(n):
        return _buf_types(n) * 2

    scratch = (
        [pltpu.VMEM((_BATCH, 4, _B), _F32) for _ in range(2)]
        + bufs(_B) + bufs(_B)
        + [pltpu.VMEM((_BATCH, 4, _TAIL_B), _F32)] + bufs(_TAIL_B)
        + [pltpu.SemaphoreType.DMA] * 4
    )
    mesh = plsc.VectorSubcoreMesh(
        core_axis_name="c", subcore_axis_name="s", num_cores=_NC, num_subcores=_NS)
    return pl.kernel(
        _sc_body, out_type=out_type, mesh=mesh, scratch_types=scratch,
        compiler_params=pltpu.CompilerParams(needs_layout_passes=False),
        name="pc_preprocessor3d_sc",
    )(pct)


def kernel(pc):
    pct = jnp.transpose(pc, (0, 2, 1))
    (pct_echo, idx0, xy0, top0, idx20, xz0, front0,
     idx1, xy1, top1, idx21, xz1, front1) = _run_sc(pct)
    tr = lambda a: jnp.transpose(a, (0, 2, 1))
    trc = lambda a: jnp.transpose(a, (1, 2, 0))
    return (
        tr(pct_echo),
        idx0, tr(xy0), tr(top0), idx20, trc(xz0), tr(front0),
        idx1, tr(xy1), tr(top1), idx21, trc(xz1), tr(front1),
    )


# speedup vs baseline: 1.0150x; 1.0150x over previous
import numpy as np
import jax
import jax.numpy as jnp
from jax import lax
from jax.experimental import pallas as pl
from jax.experimental.pallas import tpu as pltpu
from jax.experimental.pallas import tpu_sc as plsc

_BATCH = 4
_NPTS = 120000
_L = 16
_NC, _NS = 2, 16
_NW = _NC * _NS
_B = 384
_NBLK = _NPTS // _B
_TAIL_S = _NBLK * _B
_TAIL_B = _NPTS - _TAIL_S

_F32 = jnp.float32
_I32 = jnp.int32

_CX1 = np.float32(np.float32(200.0) * (np.float32(1.0) / np.float32(100.0)))
_CZ1 = np.float32(np.float32(30.0) * (np.float32(1.0) / np.float32(6.0)))


def _quant_vec(in_v, outs_v, b, j):
    (idx0_v, xy0_v, top0_v, idx20_v, xz0_v, front0_v,
     idx1_v, xy1_v, top1_v, idx21_v, xz1_v, front1_v) = outs_v
    sl = pl.ds(j * _L, _L)
    xv = in_v[b, 0, sl]
    yv = in_v[b, 1, sl]
    zv = in_v[b, 2, sl]
    fx1 = (xv + 50.0) * _CX1
    fy1 = (yv + 50.0) * _CX1
    fz1 = (zv + 4.0) * _CZ1
    fx0 = fx1 + fx1
    fy0 = fy1 + fy1
    fz0 = fz1 + fz1

    def emit(fx, fy, fz, sxy, sz, idx_v, xy_v, top_v, idx2_v, xz_v, front_v):
        ix = fx.astype(_I32)
        iy = fy.astype(_I32)
        iz = fz.astype(_I32)
        rx = fx - ix.astype(_F32)
        ry = fy - iy.astype(_F32)
        rz = fz - iz.astype(_F32)
        idx_v[b, sl] = ix * sxy + iy
        idx2_v[b, sl] = ix * (sxy * sz) + iy * sz + iz
        xy_v[b, 0, sl] = ix
        xy_v[b, 1, sl] = iy
        top_v[b, 0, sl] = rx
        top_v[b, 1, sl] = ry
        xz_v[0, b, sl] = ix
        xz_v[1, b, sl] = iy
        xz_v[2, b, sl] = iz
        front_v[b, 0, sl] = rx
        front_v[b, 1, sl] = rz

    emit(fx0, fy0, fz0, 400, 60, idx0_v, xy0_v, top0_v, idx20_v, xz0_v, front0_v)
    emit(fx1, fy1, fz1, 200, 30, idx1_v, xy1_v, top1_v, idx21_v, xz1_v, front1_v)


def _compute_block(in_v, outs_v, nvec):
    def it(j, c):
        for b in range(_BATCH):
            _quant_vec(in_v, outs_v, b, j)
        return c

    lax.fori_loop(0, nvec, it, None)


def _out_copies(pct_echo_hbm, outs_hbm, in_v, outs_v, sem, s, size):
    sl = pl.ds(s, size)
    copies = [pltpu.make_async_copy(in_v, pct_echo_hbm.at[:, :, sl], sem)]
    for ov, oh in zip(outs_v, outs_hbm):
        if oh.ndim == 2:
            copies.append(pltpu.make_async_copy(ov, oh.at[:, sl], sem))
        else:
            copies.append(pltpu.make_async_copy(ov, oh.at[:, :, sl], sem))
    return copies


def _sc_body(pct_hbm, *refs):
    pct_echo_hbm = refs[0]
    outs_hbm = refs[1:13]
    in_v = refs[13:15]
    outs_v = (refs[15:27], refs[27:39])
    in_t = refs[39]
    outs_t = refs[40:52]
    sem_in = refs[52:54]
    sem_out = refs[54:56]
    wid = lax.axis_index("s") * _NC + lax.axis_index("c")
    nblk_w = (_NBLK - 1 - wid) // _NW + 1

    def in_fire(i, p):
        s = (i * _NW + wid) * _B
        pltpu.async_copy(pct_hbm.at[:, :, pl.ds(s, _B)], in_v[p], sem_in[p])

    def out_copies(i, p):
        s = (i * _NW + wid) * _B
        return _out_copies(pct_echo_hbm, outs_hbm, in_v[p], outs_v[p],
                           sem_out[p], s, _B)

    in_fire(0, 0)

    def step(g, _):
        for p in range(2):
            i = g * 2 + p

            @pl.when(i < nblk_w)
            def _():
                @pl.when(i + 1 < nblk_w)
                def _():
                    in_fire(i + 1, 1 - p)

                @pl.when(i >= 2)
                def _():
                    for c in out_copies(i - 2, p):
                        c.wait()

                pltpu.make_async_copy(
                    pct_hbm.at[:, :, pl.ds(0, _B)], in_v[p], sem_in[p]).wait()
                _compute_block(in_v[p], outs_v[p], _B // _L)
                for c in out_copies(i, p):
                    c.start()

        return _

    lax.fori_loop(0, (_NBLK // _NW) // 2 + 1, step, None)

    for p in range(2):
        @pl.when((nblk_w >= 1) & ((nblk_w - 1) % 2 == p))
        def _():
            for c in out_copies(nblk_w - 1, p):
                c.wait()

        @pl.when((nblk_w >= 2) & ((nblk_w - 2) % 2 == p))
        def _():
            for c in out_copies(nblk_w - 2, p):
                c.wait()

    @pl.when(wid == _NW - 1)
    def _():
        sl = pl.ds(_TAIL_S, _TAIL_B)
        pltpu.async_copy(pct_hbm.at[:, :, sl], in_t, sem_in[0]).wait()
        _compute_block(in_t, outs_t, _TAIL_B // _L)
        copies = _out_copies(pct_echo_hbm, outs_hbm, in_t, outs_t,
                             sem_out[0], _TAIL_S, _TAIL_B)
        for c in copies:
            c.start()
        for c in copies:
            c.wait()


def _out_types(n):
    return [
        jax.ShapeDtypeStruct((_BATCH, n), _I32),
        jax.ShapeDtypeStruct((_BATCH, 2, n), _I32),
        jax.ShapeDtypeStruct((_BATCH, 2, n), _F32),
        jax.ShapeDtypeStruct((_BATCH, n), _I32),
        jax.ShapeDtypeStruct((3, _BATCH, n), _I32),
        jax.ShapeDtypeStruct((_BATCH, 2, n), _F32),
    ]


def _buf_types(n):
    return [
        pltpu.VMEM((_BATCH, n), _I32),
        pltpu.VMEM((_BATCH, 2, n), _I32),
        pltpu.VMEM((_BATCH, 2, n), _F32),
        pltpu.VMEM((_BATCH, n), _I32),
        pltpu.VMEM((3, _BATCH, n), _I32),
        pltpu.VMEM((_BATCH, 2, n), _F32),
    ]


def _run_sc(pct):
    out_type = ([jax.ShapeDtypeStruct((_BATCH, 4, _NPTS), _F32)]
                + _out_types(_NPTS) * 2)

    def bufs(n):
        return _buf_types(n) * 2

    scratch = (
        [pltpu.VMEM((_BATCH, 4, _B), _F32) for _ in range(2)]
        + bufs(_B) + bufs(_B)
        + [pltpu.VMEM((_BATCH, 4, _TAIL_B), _F32)] + bufs(_TAIL_B)
        + [pltpu.SemaphoreType.DMA] * 4
    )
    mesh = plsc.VectorSubcoreMesh(
        core_axis_name="c", subcore_axis_name="s", num_cores=_NC, num_subcores=_NS)
    return pl.kernel(
        _sc_body, out_type=out_type, mesh=mesh, scratch_types=scratch,
        compiler_params=pltpu.CompilerParams(needs_layout_passes=False),
        name="pc_preprocessor3d_sc",
    )(pct)


def kernel(pc):
    pct = jnp.transpose(pc, (0, 2, 1))
    (pct_echo, idx0, xy0, top0, idx20, xz0, front0,
     idx1, xy1, top1, idx21, xz1, front1) = _run_sc(pct)
    tr = lambda a: jnp.transpose(a, (0, 2, 1))
    trc = lambda a: jnp.transpose(a, (1, 2, 0))
    return (
        tr(pct_echo),
        idx0, tr(xy0), tr(top0), idx20, trc(xz0), tr(front0),
        idx1, tr(xy1), tr(top1), idx21, trc(xz1), tr(front1),
    )
